# baseline (speedup 1.0000x reference)
import functools

import jax
import jax.numpy as jnp
from jax.experimental import pallas as pl
from jax.experimental.pallas import tpu as pltpu


def _half_chain(x_ref, m1_ref, b1_ref, w2_ref, b2_ref, wf1_ref,
                bf1_ref, wf2_ref, bf2_ref, *, nb):
    x2d = x_ref[...].reshape(28 * nb, 28)
    zx = jnp.zeros((nb, 28), jnp.bfloat16)
    xp = jnp.concatenate([zx, x2d, zx], axis=0)
    xc = jnp.concatenate(
        [xp[dy * nb:(dy + 28) * nb] for dy in range(3)], axis=1)

    y = jnp.dot(xc, m1_ref[...], preferred_element_type=jnp.float32)
    y = jnp.maximum(y[:, :512], y[:, 512:])
    y = y.reshape(14, 2 * nb, 512)
    a = jnp.maximum(y[:, :nb, :], y[:, nb:, :])
    a = jnp.maximum(a.reshape(14 * nb, 512) + b1_ref[...], 0.0)
    a = a.astype(jnp.bfloat16)

    zrow = jnp.zeros((nb, 512), jnp.bfloat16)
    a3 = jnp.concatenate([zrow, a, zrow], axis=0)
    x2c = jnp.concatenate(
        [a3[dy * nb:(dy + 14) * nb] for dy in range(3)], axis=1)

    f_parts = []
    for t0, t1 in ((0, 4), (4, 8), (8, 12), (12, 14)):
        xrow = x2c[t0 * nb:t1 * nb]
        ye = jnp.dot(xrow, w2_ref[:, :512],
                     preferred_element_type=jnp.float32)
        yo = jnp.dot(xrow, w2_ref[:, 512:],
                     preferred_element_type=jnp.float32)
        ym = jnp.maximum(ye, yo).reshape((t1 - t0) // 2, 2 * nb, 512)
        fc = jnp.maximum(ym[:, :nb, :], ym[:, nb:, :])
        fc = fc.reshape(((t1 - t0) // 2) * nb, 512)
        f_parts.append(jnp.maximum(fc + b2_ref[...], 0.0))
    f2d = jnp.concatenate(f_parts, axis=0).astype(jnp.bfloat16)
    f = f2d.reshape(7, nb, 512)

    xf = jnp.concatenate([f[hp] for hp in range(7)], axis=1)
    h1 = jnp.dot(xf, wf1_ref[...], preferred_element_type=jnp.float32)
    h1 = jnp.maximum(h1 + bf1_ref[...], 0.0)

    out = jnp.dot(h1, wf2_ref[...], preferred_element_type=jnp.float32)
    return out + bf2_ref[...]


def _fused_cnn_kernel(xa_ref, xb_ref, m1_ref, b1_ref, w2_ref, b2_ref,
                      wf1_ref, bf1_ref, wf2_ref, bf2_ref, o_ref, *, nb):
    oa = _half_chain(xa_ref, m1_ref, b1_ref, w2_ref, b2_ref, wf1_ref,
                     bf1_ref, wf2_ref, bf2_ref, nb=nb)
    ob = _half_chain(xb_ref, m1_ref, b1_ref, w2_ref, b2_ref, wf1_ref,
                     bf1_ref, wf2_ref, bf2_ref, nb=nb)
    o_ref[...] = jnp.concatenate([oa, ob], axis=0)


def _prep_weights(w1, b1, w2, b2, wf1, bf1):
    f32 = jnp.float32
    w1r = w1.reshape(3, 3, 32)
    wi = jnp.arange(28)[None, :, None]
    wp = jnp.arange(16)[None, None, :]
    dx = jnp.arange(3)[:, None, None]
    valid = (wp >= 1) & (wp <= 14)
    m_e = (wi == 2 * (wp - 1) + dx - 1) & valid
    m_o = (wi == 2 * (wp - 1) + dx) & valid
    me = jnp.einsum("yxc,xiw->yiwc", w1r, m_e.astype(f32)).reshape(84, 512)
    mo = jnp.einsum("yxc,xiw->yiwc", w1r, m_o.astype(f32)).reshape(84, 512)
    m1 = jnp.concatenate([me, mo], axis=1)
    b1t = jnp.zeros((16, 32), f32).at[1:15].set(
        jnp.broadcast_to(b1.reshape(1, 32), (14, 32))).reshape(1, 512)

    w2r = w2.reshape(3, 3, 32, 64)
    wi2 = jnp.arange(16)[None, :, None]
    wp2 = jnp.arange(8)[None, None, :]
    valid2 = wp2 <= 6
    m2_e = (wi2 == 2 * wp2 + dx) & valid2
    m2_o = (wi2 == 2 * wp2 + 1 + dx) & valid2
    w2e = jnp.einsum("yxco,xiw->yicwo", w2r,
                     m2_e.astype(f32)).reshape(1536, 512)
    w2o = jnp.einsum("yxco,xiw->yicwo", w2r,
                     m2_o.astype(f32)).reshape(1536, 512)
    w2m = jnp.concatenate([w2e, w2o], axis=1)
    b2t = jnp.zeros((8, 64), f32).at[:7].set(
        jnp.broadcast_to(b2.reshape(1, 64), (7, 64))).reshape(1, 512)

    wf1p = jnp.zeros((7, 512, 128), f32).at[:, :448, :].set(
        wf1.reshape(7, 448, 128)).reshape(3584, 128)
    bf16 = jnp.bfloat16
    return m1.astype(bf16), b1t, w2m.astype(bf16), b2t, wf1p.astype(bf16)


def kernel(w1, b1, w2, b2, wf1, bf1, wf2, bf2, x):
    B = x.shape[0]
    nb = 256 if B % 512 == 0 else 8
    m1, b1t, w2m, b2t, wf1p = _prep_weights(w1, b1, w2, b2, wf1, bf1)

    xt = jnp.transpose(x.reshape(B, 28, 28).astype(jnp.bfloat16),
                       (1, 0, 2))

    body = functools.partial(_fused_cnn_kernel, nb=nb)
    return pl.pallas_call(
        body,
        out_shape=jax.ShapeDtypeStruct((B, 10), jnp.float32),
        grid=(B // (2 * nb),),
        in_specs=[
            pl.BlockSpec((28, nb, 28), lambda i: (0, 2 * i, 0)),
            pl.BlockSpec((28, nb, 28), lambda i: (0, 2 * i + 1, 0)),
            pl.BlockSpec((84, 1024), lambda i: (0, 0)),
            pl.BlockSpec((1, 512), lambda i: (0, 0)),
            pl.BlockSpec((1536, 1024), lambda i: (0, 0)),
            pl.BlockSpec((1, 512), lambda i: (0, 0)),
            pl.BlockSpec((3584, 128), lambda i: (0, 0)),
            pl.BlockSpec((1, 128), lambda i: (0, 0)),
            pl.BlockSpec((128, 10), lambda i: (0, 0)),
            pl.BlockSpec((1, 10), lambda i: (0, 0)),
        ],
        out_specs=pl.BlockSpec((2 * nb, 10), lambda i: (i, 0)),
        compiler_params=pltpu.CompilerParams(
            dimension_semantics=("arbitrary",),
            vmem_limit_bytes=100 * 1024 * 1024,
        ),
    )(xt, xt, m1, b1t, w2m, b2t, wf1p, bf1, wf2, bf2)


# speedup vs baseline: 1.0928x; 1.0928x over previous
import functools

import jax
import jax.numpy as jnp
from jax.experimental import pallas as pl
from jax.experimental.pallas import tpu as pltpu


def _half_chain(x_ref, m1_ref, b1_ref, w2_ref, b2_ref, wf1_ref,
                bf1_ref, wf2_ref, bf2_ref, *, nb):
    x2d = x_ref[...].reshape(28 * nb, 28)
    zx = jnp.zeros((nb, 28), jnp.float32)
    xp = jnp.concatenate([zx, x2d, zx], axis=0)
    xc = jnp.concatenate(
        [xp[dy * nb:(dy + 28) * nb] for dy in range(3)], axis=1)
    xc = xc.astype(jnp.bfloat16)

    y = jnp.dot(xc, m1_ref[...], preferred_element_type=jnp.float32)
    y = jnp.maximum(y[:, :512], y[:, 512:])
    y = y.reshape(14, 2 * nb, 512)
    a = jnp.maximum(y[:, :nb, :], y[:, nb:, :])
    a = jnp.maximum(a.reshape(14 * nb, 512) + b1_ref[...], 0.0)
    a = a.astype(jnp.bfloat16)

    zrow = jnp.zeros((nb, 512), jnp.bfloat16)
    a3 = jnp.concatenate([zrow, a, zrow], axis=0)
    x2c = jnp.concatenate(
        [a3[dy * nb:(dy + 14) * nb] for dy in range(3)], axis=1)

    f_parts = []
    for t0, t1 in ((0, 4), (4, 8), (8, 12), (12, 14)):
        xrow = x2c[t0 * nb:t1 * nb]
        ye = jnp.dot(xrow, w2_ref[:, :512],
                     preferred_element_type=jnp.float32)
        yo = jnp.dot(xrow, w2_ref[:, 512:],
                     preferred_element_type=jnp.float32)
        ym = jnp.maximum(ye, yo).reshape((t1 - t0) // 2, 2 * nb, 512)
        fc = jnp.maximum(ym[:, :nb, :], ym[:, nb:, :])
        fc = fc.reshape(((t1 - t0) // 2) * nb, 512)
        f_parts.append(jnp.maximum(fc + b2_ref[...], 0.0))
    f2d = jnp.concatenate(f_parts, axis=0).astype(jnp.bfloat16)
    f = f2d.reshape(7, nb, 512)

    xf = jnp.concatenate([f[hp] for hp in range(7)], axis=1)
    h1 = jnp.dot(xf, wf1_ref[...], preferred_element_type=jnp.float32)
    h1 = jnp.maximum(h1 + bf1_ref[...], 0.0)

    out = jnp.dot(h1, wf2_ref[...], preferred_element_type=jnp.float32)
    return out + bf2_ref[...]


def _fused_cnn_kernel(xa_ref, xb_ref, m1_ref, b1_ref, w2_ref, b2_ref,
                      wf1_ref, bf1_ref, wf2_ref, bf2_ref, o_ref, *, nb):
    oa = _half_chain(xa_ref, m1_ref, b1_ref, w2_ref, b2_ref, wf1_ref,
                     bf1_ref, wf2_ref, bf2_ref, nb=nb)
    ob = _half_chain(xb_ref, m1_ref, b1_ref, w2_ref, b2_ref, wf1_ref,
                     bf1_ref, wf2_ref, bf2_ref, nb=nb)
    o_ref[...] = jnp.concatenate([oa, ob], axis=0)


def _prep_weights(w1, b1, w2, b2, wf1, bf1):
    f32 = jnp.float32
    w1r = w1.reshape(3, 3, 32)
    wi = jnp.arange(28)[None, :, None]
    wp = jnp.arange(16)[None, None, :]
    dx = jnp.arange(3)[:, None, None]
    valid = (wp >= 1) & (wp <= 14)
    m_e = (wi == 2 * (wp - 1) + dx - 1) & valid
    m_o = (wi == 2 * (wp - 1) + dx) & valid
    me = jnp.einsum("yxc,xiw->yiwc", w1r, m_e.astype(f32)).reshape(84, 512)
    mo = jnp.einsum("yxc,xiw->yiwc", w1r, m_o.astype(f32)).reshape(84, 512)
    m1 = jnp.concatenate([me, mo], axis=1)
    b1t = jnp.zeros((16, 32), f32).at[1:15].set(
        jnp.broadcast_to(b1.reshape(1, 32), (14, 32))).reshape(1, 512)

    w2r = w2.reshape(3, 3, 32, 64)
    wi2 = jnp.arange(16)[None, :, None]
    wp2 = jnp.arange(8)[None, None, :]
    valid2 = wp2 <= 6
    m2_e = (wi2 == 2 * wp2 + dx) & valid2
    m2_o = (wi2 == 2 * wp2 + 1 + dx) & valid2
    w2e = jnp.einsum("yxco,xiw->yicwo", w2r,
                     m2_e.astype(f32)).reshape(1536, 512)
    w2o = jnp.einsum("yxco,xiw->yicwo", w2r,
                     m2_o.astype(f32)).reshape(1536, 512)
    w2m = jnp.concatenate([w2e, w2o], axis=1)
    b2t = jnp.zeros((8, 64), f32).at[:7].set(
        jnp.broadcast_to(b2.reshape(1, 64), (7, 64))).reshape(1, 512)

    wf1p = jnp.zeros((7, 512, 128), f32).at[:, :448, :].set(
        wf1.reshape(7, 448, 128)).reshape(3584, 128)
    bf16 = jnp.bfloat16
    return m1.astype(bf16), b1t, w2m.astype(bf16), b2t, wf1p.astype(bf16)


def kernel(w1, b1, w2, b2, wf1, bf1, wf2, bf2, x):
    B = x.shape[0]
    nb = 256 if B % 512 == 0 else 8
    m1, b1t, w2m, b2t, wf1p = _prep_weights(w1, b1, w2, b2, wf1, bf1)

    xt = jnp.transpose(x.reshape(B, 28, 28), (1, 0, 2))

    body = functools.partial(_fused_cnn_kernel, nb=nb)
    return pl.pallas_call(
        body,
        out_shape=jax.ShapeDtypeStruct((B, 10), jnp.float32),
        grid=(B // (2 * nb),),
        in_specs=[
            pl.BlockSpec((28, nb, 28), lambda i: (0, 2 * i, 0)),
            pl.BlockSpec((28, nb, 28), lambda i: (0, 2 * i + 1, 0)),
            pl.BlockSpec((84, 1024), lambda i: (0, 0)),
            pl.BlockSpec((1, 512), lambda i: (0, 0)),
            pl.BlockSpec((1536, 1024), lambda i: (0, 0)),
            pl.BlockSpec((1, 512), lambda i: (0, 0)),
            pl.BlockSpec((3584, 128), lambda i: (0, 0)),
            pl.BlockSpec((1, 128), lambda i: (0, 0)),
            pl.BlockSpec((128, 10), lambda i: (0, 0)),
            pl.BlockSpec((1, 10), lambda i: (0, 0)),
        ],
        out_specs=pl.BlockSpec((2 * nb, 10), lambda i: (i, 0)),
        compiler_params=pltpu.CompilerParams(
            dimension_semantics=("arbitrary",),
            vmem_limit_bytes=100 * 1024 * 1024,
        ),
    )(xt, xt, m1, b1t, w2m, b2t, wf1p, bf1, wf2, bf2)
